# Initial kernel scaffold; baseline (speedup 1.0000x reference)
import jax
import jax.numpy as jnp
from jax.experimental import pallas as pl


def kernel(x, edge_index, W1l, b1l, W1r, W2l, b2l, W2r):
    raise NotImplementedError("write your pallas kernel here")


# speedup vs baseline: 10.8762x; 10.8762x over previous
import functools

import jax
import jax.numpy as jnp
from jax import lax
from jax.experimental import pallas as pl
from jax.experimental.pallas import tpu as pltpu
from jax.experimental.pallas import tpu_sc as plsc

N = 10000
D = 128
H = 128
C = 64
E = 320000

NC = 2
NS = 16
NW = NC * NS
K = 125
CH = E // (NW * K)
NPAD = 10240
RPT = NPAD // NS
NBUF = 2


def _make_segsum(Dd, with_deg):
    mesh = plsc.VectorSubcoreMesh(core_axis_name="c", subcore_axis_name="s")
    out_type = [jax.ShapeDtypeStruct((NC, NPAD, Dd), jnp.float32)]
    scratch = [
        pltpu.VMEM((CH, K), jnp.int32),
        pltpu.VMEM((CH, K), jnp.int32),
        pltpu.VMEM((K, Dd), jnp.float32),
        pltpu.VMEM((K, Dd), jnp.float32),
        pltpu.VMEM_SHARED((NPAD, Dd), jnp.float32),
        pltpu.SemaphoreType.DMA,
        pltpu.SemaphoreType.DMA,
    ]
    if with_deg:
        out_type.append(jax.ShapeDtypeStruct((NC, NPAD), jnp.float32))
        scratch += [
            pltpu.VMEM((K,), jnp.float32),
            pltpu.VMEM_SHARED((NPAD,), jnp.float32),
        ]

    @functools.partial(
        pl.kernel, mesh=mesh, out_type=out_type, scratch_types=scratch,
        compiler_params=pltpu.CompilerParams(use_tc_tiling_on_sc=False))
    def seg(*refs):
        if with_deg:
            (y_hbm, src_hbm, dst_hbm, zeros_hbm, zdeg_hbm, ones_hbm,
             out_hbm, deg_hbm,
             src_v, dst_v, buf0, buf1, acc_sh, sem0, sem1,
             ones_v, deg_sh) = refs
        else:
            (y_hbm, src_hbm, dst_hbm, zeros_hbm,
             out_hbm,
             src_v, dst_v, buf0, buf1, acc_sh, sem0, sem1) = refs
        c = lax.axis_index("c")
        s = lax.axis_index("s")
        wid = s * NC + c
        bufs = (buf0, buf1)
        sems = (sem0, sem1)

        pltpu.sync_copy(src_hbm.at[wid], src_v)
        pltpu.sync_copy(dst_hbm.at[wid], dst_v)
        for b in range(NBUF):
            pltpu.async_copy(y_hbm.at[src_v.at[b]], bufs[b], sems[b])

        r0 = s * RPT
        pltpu.sync_copy(zeros_hbm.at[pl.ds(r0, RPT)],
                        acc_sh.at[pl.ds(r0, RPT)])
        if with_deg:
            pltpu.sync_copy(zdeg_hbm.at[pl.ds(r0, RPT)],
                            deg_sh.at[pl.ds(r0, RPT)])
            pltpu.sync_copy(ones_hbm, ones_v)
        plsc.subcore_barrier()

        def chunk_pair(t, carry):
            for b in range(NBUF):
                ch = t * NBUF + b
                pltpu.make_async_copy(y_hbm.at[src_v.at[ch]], bufs[b],
                                      sems[b]).wait()
                pltpu.sync_copy(bufs[b], acc_sh.at[dst_v.at[ch]], add=True)
                if with_deg:
                    pltpu.sync_copy(ones_v, deg_sh.at[dst_v.at[ch]],
                                    add=True)
                nxt = ch + NBUF

                @pl.when(nxt < CH)
                def _():
                    pltpu.async_copy(y_hbm.at[src_v.at[nxt]], bufs[b],
                                     sems[b])
            return carry

        lax.fori_loop(0, CH // NBUF, chunk_pair, 0)
        plsc.subcore_barrier()

        pltpu.sync_copy(acc_sh.at[pl.ds(r0, RPT)],
                        out_hbm.at[c, pl.ds(r0, RPT)])
        if with_deg:
            pltpu.sync_copy(deg_sh.at[pl.ds(r0, RPT)],
                            deg_hbm.at[c, pl.ds(r0, RPT)])

    return seg


_segsum_deg = _make_segsum(C, True)
_segsum = _make_segsum(C, False)

BN = 400


def _mm3_body(x_ref, wa_ref, wb_ref, wc_ref, ya_ref, yb_ref, yc_ref):
    x = x_ref[...]
    ya_ref[...] = jnp.dot(x, wa_ref[...], preferred_element_type=jnp.float32)
    yb_ref[...] = jnp.dot(x, wb_ref[...], preferred_element_type=jnp.float32)
    yc_ref[...] = jnp.dot(x, wc_ref[...], preferred_element_type=jnp.float32)


def _mm3(x, wa, wb, wc):
    d = x.shape[1]
    return pl.pallas_call(
        _mm3_body,
        grid=(N // BN,),
        in_specs=[pl.BlockSpec((BN, d), lambda i: (i, 0)),
                  pl.BlockSpec(wa.shape, lambda i: (0, 0)),
                  pl.BlockSpec(wb.shape, lambda i: (0, 0)),
                  pl.BlockSpec(wc.shape, lambda i: (0, 0))],
        out_specs=[pl.BlockSpec((BN, wa.shape[1]), lambda i: (i, 0)),
                   pl.BlockSpec((BN, wb.shape[1]), lambda i: (i, 0)),
                   pl.BlockSpec((BN, wc.shape[1]), lambda i: (i, 0))],
        out_shape=[jax.ShapeDtypeStruct((N, wa.shape[1]), jnp.float32),
                   jax.ShapeDtypeStruct((N, wb.shape[1]), jnp.float32),
                   jax.ShapeDtypeStruct((N, wc.shape[1]), jnp.float32)],
    )(x, wa, wb, wc)


def _mid_body(s1pa_ref, s1pb_ref, deg_ref, r1_ref, b1l_ref, w2l_ref, w2r_ref,
              y2_ref, z2_ref):
    ssum = jnp.concatenate([s1pa_ref[0] + s1pa_ref[1],
                            s1pb_ref[0] + s1pb_ref[1]], axis=-1)
    deg = jnp.maximum(deg_ref[...], 1.0)
    o = ssum / deg + b1l_ref[...] + r1_ref[...]
    nrm = jnp.sqrt(jnp.sum(o * o, axis=-1, keepdims=True))
    h = jnp.maximum(o / jnp.maximum(nrm, 1e-12), 0.0)
    y2_ref[...] = jnp.dot(h, w2l_ref[...], preferred_element_type=jnp.float32)
    z2_ref[...] = jnp.dot(h, w2r_ref[...], preferred_element_type=jnp.float32)


def _mid(s1pa, s1pb, deg, r1, b1l, w2l, w2r):
    return pl.pallas_call(
        _mid_body,
        grid=(N // BN,),
        in_specs=[pl.BlockSpec((NC, BN, C), lambda i: (0, i, 0)),
                  pl.BlockSpec((NC, BN, C), lambda i: (0, i, 0)),
                  pl.BlockSpec((BN, 1), lambda i: (i, 0)),
                  pl.BlockSpec((BN, H), lambda i: (i, 0)),
                  pl.BlockSpec((1, H), lambda i: (0, 0)),
                  pl.BlockSpec((H, C), lambda i: (0, 0)),
                  pl.BlockSpec((H, C), lambda i: (0, 0))],
        out_specs=[pl.BlockSpec((BN, C), lambda i: (i, 0)),
                   pl.BlockSpec((BN, C), lambda i: (i, 0))],
        out_shape=[jax.ShapeDtypeStruct((N, C), jnp.float32),
                   jax.ShapeDtypeStruct((N, C), jnp.float32)],
    )(s1pa, s1pb, deg, r1, b1l, w2l, w2r)


def _fin_body(s2p_ref, deg_ref, z2_ref, b2l_ref, out_ref):
    ssum = s2p_ref[0] + s2p_ref[1]
    deg = jnp.maximum(deg_ref[...], 1.0)
    o = ssum / deg + b2l_ref[...] + z2_ref[...]
    nrm = jnp.sqrt(jnp.sum(o * o, axis=-1, keepdims=True))
    o = o / jnp.maximum(nrm, 1e-12)
    m = jnp.max(o, axis=-1, keepdims=True)
    e = jnp.exp(o - m)
    lse = jnp.log(jnp.sum(e, axis=-1, keepdims=True))
    out_ref[...] = o - m - lse


def _fin(s2p, deg, z2, b2l):
    return pl.pallas_call(
        _fin_body,
        grid=(N // BN,),
        in_specs=[pl.BlockSpec((NC, BN, C), lambda i: (0, i, 0)),
                  pl.BlockSpec((BN, 1), lambda i: (i, 0)),
                  pl.BlockSpec((BN, C), lambda i: (i, 0)),
                  pl.BlockSpec((1, C), lambda i: (0, 0))],
        out_specs=pl.BlockSpec((BN, C), lambda i: (i, 0)),
        out_shape=jax.ShapeDtypeStruct((N, C), jnp.float32),
    )(s2p, deg, z2, b2l)


def kernel(x, edge_index, W1l, b1l, W1r, W2l, b2l, W2r):
    src3 = edge_index[0].reshape(NW, CH, K)
    dst3 = edge_index[1].reshape(NW, CH, K)
    zeros_c = jnp.zeros((NPAD, C), jnp.float32)
    zdeg = jnp.zeros((NPAD,), jnp.float32)
    ones = jnp.ones((K,), jnp.float32)

    y1a, y1b, r1 = _mm3(x, W1l[:, :C], W1l[:, C:], W1r)
    s1pa, degp = _segsum_deg(y1a, src3, dst3, zeros_c, zdeg, ones)
    s1pb = _segsum(y1b, src3, dst3, zeros_c)
    if isinstance(s1pb, (list, tuple)):
        s1pb = s1pb[0]
    deg = (degp[0, :N] + degp[1, :N]).reshape(N, 1)
    y2, z2 = _mid(s1pa, s1pb, deg, r1, b1l.reshape(1, H), W2l, W2r)
    s2p = _segsum(y2, src3, dst3, zeros_c)
    if isinstance(s2p, (list, tuple)):
        s2p = s2p[0]
    return _fin(s2p, deg, z2, b2l.reshape(1, C))


# speedup vs baseline: 12.1043x; 1.1129x over previous
import functools

import jax
import jax.numpy as jnp
from jax import lax
from jax.experimental import pallas as pl
from jax.experimental.pallas import tpu as pltpu
from jax.experimental.pallas import tpu_sc as plsc

N = 10000
D = 128
H = 128
C = 64
E = 320000

NC = 2
NS = 16
NW = NC * NS
K = 125
CH = E // (NW * K)
NPAD = 10240
RPT = NPAD // NS
NBUF = 8
LEAD = 4

_SC_PARAMS = pltpu.CompilerParams(use_tc_tiling_on_sc=False)
_MESH = plsc.VectorSubcoreMesh(core_axis_name="c", subcore_axis_name="s")


def _edge_ring(table, acc_sh, src_v, dst_v, bufs, gsems, ssems, deg=None):
    for b in range(LEAD):
        pltpu.async_copy(table.at[src_v.at[b]], bufs[b], gsems[b])

    def outer(t, carry):
        for b in range(NBUF):
            ch = t * NBUF + b
            g = ch + LEAD
            bg = (b + LEAD) % NBUF

            @pl.when(g < CH)
            def _():
                @pl.when(g >= NBUF)
                def _():
                    pltpu.make_async_copy(
                        bufs[bg], acc_sh.at[dst_v.at[g - NBUF]],
                        ssems[bg]).wait()

                pltpu.async_copy(table.at[src_v.at[g]], bufs[bg], gsems[bg])

            pltpu.make_async_copy(table.at[src_v.at[ch]], bufs[b],
                                  gsems[b]).wait()
            pltpu.async_copy(bufs[b], acc_sh.at[dst_v.at[ch]], ssems[b],
                             add=True)
            if deg is not None:
                ones_v, deg_sh, degsem = deg
                pltpu.async_copy(ones_v, deg_sh.at[dst_v.at[ch]], degsem,
                                 add=True)
        return carry

    lax.fori_loop(0, CH // NBUF, outer, 0)
    for b in range(NBUF):
        ch = CH - NBUF + b
        pltpu.make_async_copy(bufs[b], acc_sh.at[dst_v.at[ch]],
                              ssems[b]).wait()
    if deg is not None:
        ones_v, deg_sh, degsem = deg

        def drain(i, carry):
            pltpu.make_async_copy(ones_v, deg_sh.at[dst_v.at[0]],
                                  degsem).wait()
            return carry

        lax.fori_loop(0, CH, drain, 0)


def _make_seg1():
    out_type = [jax.ShapeDtypeStruct((NC, NPAD, C), jnp.float32),
                jax.ShapeDtypeStruct((NC, NPAD, C), jnp.float32),
                jax.ShapeDtypeStruct((NC, NPAD), jnp.float32)]
    scratch = (
        [pltpu.VMEM((CH, K), jnp.int32), pltpu.VMEM((CH, K), jnp.int32)]
        + [pltpu.VMEM((K, C), jnp.float32)] * NBUF
        + [pltpu.VMEM_SHARED((NPAD, C), jnp.float32)]
        + [pltpu.SemaphoreType.DMA] * (2 * NBUF)
        + [pltpu.VMEM((K,), jnp.float32),
           pltpu.VMEM_SHARED((NPAD,), jnp.float32),
           pltpu.SemaphoreType.DMA]
    )

    @functools.partial(pl.kernel, mesh=_MESH, out_type=out_type,
                       scratch_types=scratch, compiler_params=_SC_PARAMS)
    def seg1(ya_hbm, yb_hbm, src_hbm, dst_hbm, zeros_hbm, zdeg_hbm,
             ones_hbm, outa_hbm, outb_hbm, dego_hbm, src_v, dst_v, *rest):
        bufs = rest[:NBUF]
        acc_sh = rest[NBUF]
        gsems = rest[NBUF + 1:2 * NBUF + 1]
        ssems = rest[2 * NBUF + 1:3 * NBUF + 1]
        ones_v, deg_sh, degsem = rest[3 * NBUF + 1:]
        c = lax.axis_index("c")
        s = lax.axis_index("s")
        wid = s * NC + c
        r0 = s * RPT

        pltpu.sync_copy(src_hbm.at[wid], src_v)
        pltpu.sync_copy(dst_hbm.at[wid], dst_v)
        pltpu.sync_copy(ones_hbm, ones_v)
        pltpu.sync_copy(zeros_hbm.at[pl.ds(r0, RPT)],
                        acc_sh.at[pl.ds(r0, RPT)])
        pltpu.sync_copy(zdeg_hbm.at[pl.ds(r0, RPT)],
                        deg_sh.at[pl.ds(r0, RPT)])
        plsc.subcore_barrier()

        _edge_ring(ya_hbm, acc_sh, src_v, dst_v, bufs, gsems, ssems,
                   deg=(ones_v, deg_sh, degsem))
        plsc.subcore_barrier()
        pltpu.sync_copy(acc_sh.at[pl.ds(r0, RPT)],
                        outa_hbm.at[c, pl.ds(r0, RPT)])
        pltpu.sync_copy(deg_sh.at[pl.ds(r0, RPT)],
                        dego_hbm.at[c, pl.ds(r0, RPT)])
        pltpu.sync_copy(zeros_hbm.at[pl.ds(r0, RPT)],
                        acc_sh.at[pl.ds(r0, RPT)])
        plsc.subcore_barrier()

        _edge_ring(yb_hbm, acc_sh, src_v, dst_v, bufs, gsems, ssems)
        plsc.subcore_barrier()
        pltpu.sync_copy(acc_sh.at[pl.ds(r0, RPT)],
                        outb_hbm.at[c, pl.ds(r0, RPT)])

    return seg1


def _make_seg2():
    out_type = [jax.ShapeDtypeStruct((NC, NPAD, C), jnp.float32)]
    scratch = (
        [pltpu.VMEM((CH, K), jnp.int32), pltpu.VMEM((CH, K), jnp.int32)]
        + [pltpu.VMEM((K, C), jnp.float32)] * NBUF
        + [pltpu.VMEM_SHARED((NPAD, C), jnp.float32)]
        + [pltpu.SemaphoreType.DMA] * (2 * NBUF)
    )

    @functools.partial(pl.kernel, mesh=_MESH, out_type=out_type,
                       scratch_types=scratch, compiler_params=_SC_PARAMS)
    def seg2(y_hbm, src_hbm, dst_hbm, zeros_hbm, out_hbm,
             src_v, dst_v, *rest):
        bufs = rest[:NBUF]
        acc_sh = rest[NBUF]
        gsems = rest[NBUF + 1:2 * NBUF + 1]
        ssems = rest[2 * NBUF + 1:3 * NBUF + 1]
        c = lax.axis_index("c")
        s = lax.axis_index("s")
        wid = s * NC + c
        r0 = s * RPT

        pltpu.sync_copy(src_hbm.at[wid], src_v)
        pltpu.sync_copy(dst_hbm.at[wid], dst_v)
        pltpu.sync_copy(zeros_hbm.at[pl.ds(r0, RPT)],
                        acc_sh.at[pl.ds(r0, RPT)])
        plsc.subcore_barrier()

        _edge_ring(y_hbm, acc_sh, src_v, dst_v, bufs, gsems, ssems)
        plsc.subcore_barrier()
        pltpu.sync_copy(acc_sh.at[pl.ds(r0, RPT)],
                        out_hbm.at[c, pl.ds(r0, RPT)])

    return seg2


_seg1 = _make_seg1()
_seg2 = _make_seg2()

BN = 400


def _mm3_body(x_ref, wa_ref, wb_ref, wc_ref, ya_ref, yb_ref, yc_ref):
    x = x_ref[...]
    ya_ref[...] = jnp.dot(x, wa_ref[...], preferred_element_type=jnp.float32)
    yb_ref[...] = jnp.dot(x, wb_ref[...], preferred_element_type=jnp.float32)
    yc_ref[...] = jnp.dot(x, wc_ref[...], preferred_element_type=jnp.float32)


def _mm3(x, wa, wb, wc):
    d = x.shape[1]
    return pl.pallas_call(
        _mm3_body,
        grid=(N // BN,),
        in_specs=[pl.BlockSpec((BN, d), lambda i: (i, 0)),
                  pl.BlockSpec(wa.shape, lambda i: (0, 0)),
                  pl.BlockSpec(wb.shape, lambda i: (0, 0)),
                  pl.BlockSpec(wc.shape, lambda i: (0, 0))],
        out_specs=[pl.BlockSpec((BN, wa.shape[1]), lambda i: (i, 0)),
                   pl.BlockSpec((BN, wb.shape[1]), lambda i: (i, 0)),
                   pl.BlockSpec((BN, wc.shape[1]), lambda i: (i, 0))],
        out_shape=[jax.ShapeDtypeStruct((N, wa.shape[1]), jnp.float32),
                   jax.ShapeDtypeStruct((N, wb.shape[1]), jnp.float32),
                   jax.ShapeDtypeStruct((N, wc.shape[1]), jnp.float32)],
    )(x, wa, wb, wc)


def _mid_body(s1pa_ref, s1pb_ref, deg_ref, r1_ref, b1l_ref, w2l_ref, w2r_ref,
              y2_ref, z2_ref):
    ssum = jnp.concatenate([s1pa_ref[0] + s1pa_ref[1],
                            s1pb_ref[0] + s1pb_ref[1]], axis=-1)
    deg = jnp.maximum(deg_ref[...], 1.0)
    o = ssum / deg + b1l_ref[...] + r1_ref[...]
    nrm = jnp.sqrt(jnp.sum(o * o, axis=-1, keepdims=True))
    h = jnp.maximum(o / jnp.maximum(nrm, 1e-12), 0.0)
    y2_ref[...] = jnp.dot(h, w2l_ref[...], preferred_element_type=jnp.float32)
    z2_ref[...] = jnp.dot(h, w2r_ref[...], preferred_element_type=jnp.float32)


def _mid(s1pa, s1pb, deg, r1, b1l, w2l, w2r):
    return pl.pallas_call(
        _mid_body,
        grid=(N // BN,),
        in_specs=[pl.BlockSpec((NC, BN, C), lambda i: (0, i, 0)),
                  pl.BlockSpec((NC, BN, C), lambda i: (0, i, 0)),
                  pl.BlockSpec((BN, 1), lambda i: (i, 0)),
                  pl.BlockSpec((BN, H), lambda i: (i, 0)),
                  pl.BlockSpec((1, H), lambda i: (0, 0)),
                  pl.BlockSpec((H, C), lambda i: (0, 0)),
                  pl.BlockSpec((H, C), lambda i: (0, 0))],
        out_specs=[pl.BlockSpec((BN, C), lambda i: (i, 0)),
                   pl.BlockSpec((BN, C), lambda i: (i, 0))],
        out_shape=[jax.ShapeDtypeStruct((N, C), jnp.float32),
                   jax.ShapeDtypeStruct((N, C), jnp.float32)],
    )(s1pa, s1pb, deg, r1, b1l, w2l, w2r)


def _fin_body(s2p_ref, deg_ref, z2_ref, b2l_ref, out_ref):
    ssum = s2p_ref[0] + s2p_ref[1]
    deg = jnp.maximum(deg_ref[...], 1.0)
    o = ssum / deg + b2l_ref[...] + z2_ref[...]
    nrm = jnp.sqrt(jnp.sum(o * o, axis=-1, keepdims=True))
    o = o / jnp.maximum(nrm, 1e-12)
    m = jnp.max(o, axis=-1, keepdims=True)
    e = jnp.exp(o - m)
    lse = jnp.log(jnp.sum(e, axis=-1, keepdims=True))
    out_ref[...] = o - m - lse


def _fin(s2p, deg, z2, b2l):
    return pl.pallas_call(
        _fin_body,
        grid=(N // BN,),
        in_specs=[pl.BlockSpec((NC, BN, C), lambda i: (0, i, 0)),
                  pl.BlockSpec((BN, 1), lambda i: (i, 0)),
                  pl.BlockSpec((BN, C), lambda i: (i, 0)),
                  pl.BlockSpec((1, C), lambda i: (0, 0))],
        out_specs=pl.BlockSpec((BN, C), lambda i: (i, 0)),
        out_shape=jax.ShapeDtypeStruct((N, C), jnp.float32),
    )(s2p, deg, z2, b2l)


def kernel(x, edge_index, W1l, b1l, W1r, W2l, b2l, W2r):
    src3 = edge_index[0].reshape(NW, CH, K)
    dst3 = edge_index[1].reshape(NW, CH, K)
    zeros_c = jnp.zeros((NPAD, C), jnp.float32)
    zdeg = jnp.zeros((NPAD,), jnp.float32)
    ones = jnp.ones((K,), jnp.float32)

    y1a, y1b, r1 = _mm3(x, W1l[:, :C], W1l[:, C:], W1r)
    s1pa, s1pb, degp = _seg1(y1a, y1b, src3, dst3, zeros_c, zdeg, ones)
    deg = (degp[0, :N] + degp[1, :N]).reshape(N, 1)
    y2, z2 = _mid(s1pa, s1pb, deg, r1, b1l.reshape(1, H), W2l, W2r)
    s2p = _seg2(y2, src3, dst3, zeros_c)
    if isinstance(s2p, (list, tuple)):
        s2p = s2p[0]
    return _fin(s2p, deg, z2, b2l.reshape(1, C))
